# Initial kernel scaffold; baseline (speedup 1.0000x reference)
import jax
import jax.numpy as jnp
from jax.experimental import pallas as pl


def kernel(input_ids, emb_table, W1, b1, W2, b2):
    raise NotImplementedError("write your pallas kernel here")


# speedup vs baseline: 1.9197x; 1.9197x over previous
import functools

import jax
import jax.numpy as jnp
from jax import lax
from jax.experimental import pallas as pl
from jax.experimental.pallas import tpu as pltpu
from jax.experimental.pallas import tpu_sc as plsc

VOCAB = 32000
EMBED_DIM = 128
HIDDEN = 256
N_DOMAINS = 5
BATCH = 4096
SEQ = 200

NC = 2
NS = 16
NW = NC * NS
LANES = 16
B_PER_W = BATCH // NW
HALF = SEQ // 2
HALF_PAD = 104
CHUNKS_PER_W = 2 * B_PER_W
NVEC = EMBED_DIM // LANES
UNROLL = 10

_mesh = plsc.VectorSubcoreMesh(core_axis_name="c", subcore_axis_name="s")


def _accum(buf, acc):
  def step(i, acc):
    for u in range(UNROLL):
      s = i * UNROLL + u
      acc = tuple(acc[j] + buf[s, pl.ds(LANES * j, LANES)] for j in range(NVEC))
    return acc
  return lax.fori_loop(0, HALF // UNROLL, step, acc)


@functools.partial(
    pl.kernel,
    out_type=jax.ShapeDtypeStruct((BATCH, EMBED_DIM), jnp.float32),
    mesh=_mesh,
    scratch_types=[
        pltpu.VMEM((CHUNKS_PER_W, HALF_PAD), jnp.int32),
        pltpu.VMEM((HALF_PAD, EMBED_DIM), jnp.float32),
        pltpu.VMEM((HALF_PAD, EMBED_DIM), jnp.float32),
        pltpu.VMEM((HALF_PAD, EMBED_DIM), jnp.float32),
        pltpu.VMEM((HALF_PAD, EMBED_DIM), jnp.float32),
        pltpu.VMEM((B_PER_W, EMBED_DIM), jnp.float32),
        pltpu.SemaphoreType.DMA,
        pltpu.SemaphoreType.DMA,
        pltpu.SemaphoreType.DMA,
        pltpu.SemaphoreType.DMA,
    ],
)
def _pool_kernel(ids_hbm, table_hbm, out_hbm,
                 idx_v, g0, g1, g2, g3, out_v, s0, s1, s2, s3):
  wid = lax.axis_index("s") * NC + lax.axis_index("c")
  cbase = wid * CHUNKS_PER_W
  obase = wid * B_PER_W

  pltpu.sync_copy(ids_hbm.at[pl.ds(cbase, CHUNKS_PER_W)], idx_v)

  bufs = (g0, g1, g2, g3)
  sems = (s0, s1, s2, s3)

  def issue(c, buf, sem):
    pltpu.async_copy(table_hbm.at[idx_v.at[c]], buf, sem)

  def wait(buf, sem):
    pltpu.make_async_copy(table_hbm.at[idx_v.at[0]], buf, sem).wait()

  for k in range(4):
    issue(k, bufs[k], sems[k])

  zeros = tuple(jnp.zeros((LANES,), jnp.float32) for _ in range(NVEC))
  inv_s = jnp.float32(1.0 / SEQ)

  def body(i, carry):
    c = 4 * i
    for half in range(2):
      row = 2 * i + half
      acc = zeros
      for k in range(2):
        b = bufs[2 * half + k]
        sm = sems[2 * half + k]
        wait(b, sm)
        acc = _accum(b, acc)
        nxt = c + 4 + 2 * half + k

        @pl.when(nxt < CHUNKS_PER_W)
        def _():
          issue(nxt, b, sm)

      for j in range(NVEC):
        out_v[row, pl.ds(LANES * j, LANES)] = acc[j] * inv_s
    return carry

  lax.fori_loop(0, B_PER_W // 2, body, jnp.int32(0))

  pltpu.sync_copy(out_v, out_hbm.at[pl.ds(obase, B_PER_W)])


def _mlp_body(x_ref, w1_ref, b1_ref, w2_ref, b2_ref, o_ref):
  h = jnp.dot(x_ref[...], w1_ref[...], preferred_element_type=jnp.float32)
  h = jnp.maximum(h + b1_ref[...], 0.0)
  o_ref[...] = (
      jnp.dot(h, w2_ref[...], preferred_element_type=jnp.float32) + b2_ref[...]
  )


def _mlp(pooled, W1, b1, W2p, b2p):
  return pl.pallas_call(
      _mlp_body,
      out_shape=jax.ShapeDtypeStruct((BATCH, 128), jnp.float32),
  )(pooled, W1, b1, W2p, b2p)


def kernel(input_ids, emb_table, W1, b1, W2, b2):
  ids = input_ids.astype(jnp.int32).reshape(2 * BATCH, HALF)
  ids = jnp.pad(ids, ((0, 0), (0, HALF_PAD - HALF)))
  pooled = _pool_kernel(ids, emb_table)
  W2p = jnp.pad(W2, ((0, 0), (0, 128 - N_DOMAINS)))
  b2p = jnp.pad(b2, (0, 128 - N_DOMAINS)).reshape(1, 128)
  logits = _mlp(pooled, W1, b1.reshape(1, HIDDEN), W2p, b2p)
  return logits[:, :N_DOMAINS]
